# Initial kernel scaffold; baseline (speedup 1.0000x reference)
import jax
import jax.numpy as jnp
from jax.experimental import pallas as pl


def kernel(x, edge_index, edge_attr, W_fc, b_fc, W_lin, att_src, att_dst, W_edge, att_edge, bias_conv, W_att, b_att, W_out, b_out):
    raise NotImplementedError("write your pallas kernel here")


# speedup vs baseline: 49.0156x; 49.0156x over previous
import functools

import jax
import jax.numpy as jnp
from jax import lax
from jax.experimental import pallas as pl
from jax.experimental.pallas import tpu as pltpu
from jax.experimental.pallas import tpu_sc as plsc

N = 10000
E = 320000
IN_F = 128
HID = 16
HEADS = 8
EDGE_DIM = 16
OUT_F = 128

NC = 2
NS = 16
NW = NC * NS
EPT = E // NW
G = 80
NCH = EPT // G
ROW_STRIDE = 624
ROW_COPIES = 8


def _prep_nodes_body(x_ref, wfct_ref, bfc_ref, wlint_ref, cs_ref, cd_ref,
                     xt_ref, asrc_ref, adst_ref):
  h = jnp.dot(x_ref[...], wfct_ref[...], preferred_element_type=jnp.float32)
  h = h + bfc_ref[...]
  xt = jnp.dot(h, wlint_ref[...], preferred_element_type=jnp.float32)
  xt_ref[...] = xt
  asrc_ref[...] = jnp.dot(xt, cs_ref[...], preferred_element_type=jnp.float32)
  adst_ref[...] = jnp.dot(xt, cd_ref[...], preferred_element_type=jnp.float32)


def _prep_nodes(x, wfct, bfc, wlint, cs, cd):
  return pl.pallas_call(
      _prep_nodes_body,
      out_shape=[
          jax.ShapeDtypeStruct((N, HEADS * HID), jnp.float32),
          jax.ShapeDtypeStruct((N, 16), jnp.float32),
          jax.ShapeDtypeStruct((N, 16), jnp.float32),
      ],
  )(x, wfct, bfc, wlint, cs, cd)


_EB = 4000


def _prep_edges_body(ea_ref, ae_ref, out_ref):
  out_ref[...] = jnp.dot(ea_ref[...], ae_ref[...],
                         preferred_element_type=jnp.float32)


def _prep_edges(edge_attr, ae_w):
  return pl.pallas_call(
      _prep_edges_body,
      grid=(E // _EB,),
      in_specs=[
          pl.BlockSpec((_EB, EDGE_DIM), lambda i: (i, 0)),
          pl.BlockSpec((EDGE_DIM, 16), lambda i: (0, 0)),
      ],
      out_specs=pl.BlockSpec((_EB, 16), lambda i: (i, 0)),
      out_shape=jax.ShapeDtypeStruct((E, 16), jnp.float32),
  )(edge_attr, ae_w)


def _sc_body(src_hbm, dst_hbm, ae_hbm, asrc_hbm, adst_hbm, xt_hbm,
             outp_hbm, denp_hbm, attp_hbm,
             srcb, dstb, sdstb, aeb, asb, adb, xtb, exb,
             out_sh, den_sh,
             semL, semG, semS):
  cid = lax.axis_index("c")
  sid = lax.axis_index("s")
  wid = cid * NS + sid
  tile_base = wid * EPT

  io16 = lax.iota(jnp.int32, 16)
  zero16 = jnp.zeros((16,), jnp.float32)
  slope16 = jnp.full((16,), 0.2, jnp.float32)
  l0f = jnp.maximum(1.0 - io16.astype(jnp.float32),
                    jnp.zeros((16,), jnp.float32))

  def bc16(v):
    return jnp.broadcast_to(v, (16,))

  def lin_copies(ci, b):
    base = tile_base + ci * G
    return (
        pltpu.make_async_copy(src_hbm.at[pl.ds(base, G)],
                              srcb[b].at[pl.ds(0, G)], semL[b]),
        pltpu.make_async_copy(dst_hbm.at[pl.ds(base, G)],
                              dstb[b].at[pl.ds(0, G)], semL[b]),
        pltpu.make_async_copy(ae_hbm.at[pl.ds(base, G)], aeb[b], semL[b]),
    )

  def gat_copies(b):
    si = srcb[b].at[pl.ds(0, G)]
    di = dstb[b].at[pl.ds(0, G)]
    return (
        pltpu.make_async_copy(asrc_hbm.at[si], asb[b], semG[b]),
        pltpu.make_async_copy(adst_hbm.at[di], adb[b], semG[b]),
        pltpu.make_async_copy(xt_hbm.at[si], xtb[b], semG[b]),
    )

  def sca_copies(b):
    return (
        pltpu.make_async_copy(exb[b], den_sh.at[sdstb[b]], semS[b]),
        pltpu.make_async_copy(xtb[b], out_sh.at[sdstb[b]], semS[b]),
    )

  def issue(copies):
    for c in copies:
      c.start()

  def wait(copies):
    for c in copies:
      c.wait()

  def issue_scatters(b):
    def cp(t, _):
      sdstb[b][pl.ds(t * 16, 16)] = dstb[b][pl.ds(t * 16, 16)]
      return 0
    lax.fori_loop(0, G // 16, cp, 0, unroll=True)
    pltpu.async_copy(exb[b], den_sh.at[sdstb[b]], semS[b], add=True)
    pltpu.async_copy(xtb[b], out_sh.at[sdstb[b]], semS[b], add=True)

  def stage_a(b, acc):
    def body(e, acc):
      s = asb[b][e, :]
      d = adb[b][e, :]
      a = aeb[b][e, :]
      al = s + d + a
      al = jnp.maximum(al, slope16 * al)
      ex = jnp.exp(al)
      exb[b][e, :] = ex
      sv = srcb[b][pl.ds(e, 16)]
      dv = dstb[b][pl.ds(e, 16)]
      hit = (sv[0] == 6) & (dv[0] == 6)
      hitf = jnp.where(hit, jnp.float32(1.0), jnp.float32(0.0))
      return acc + ex * l0f * bc16(hitf)
    return lax.fori_loop(0, G, body, acc, unroll=4)

  def stage_b(b):
    def body(e, _):
      er = exb[b][e, :]
      for h in range(HEADS):
        exs = er.at[bc16(h)].get(mode="promise_in_bounds")
        sl = pl.ds(h * 16, 16)
        xtb[b][e, sl] = xtb[b][e, sl] * exs
      return 0
    lax.fori_loop(0, G, body, 0, unroll=2)

  def zero_body(i, _):
    xtb[0][lax.shift_right_logical(i, 3),
           pl.ds(jnp.bitwise_and(i, 7) * 16, 16)] = zero16
    return 0
  lax.fori_loop(0, G * HEADS, zero_body, 0, unroll=8)

  def zero_ex(e, _):
    exb[0][e, :] = zero16
    return 0
  lax.fori_loop(0, G, zero_ex, 0, unroll=4)

  row0 = sid * ROW_STRIDE
  for c in range(ROW_COPIES):
    pltpu.sync_copy(xtb[0], out_sh.at[pl.ds(row0 + c * G, G)])
    pltpu.sync_copy(exb[0], den_sh.at[pl.ds(row0 + c * G, G)])
  plsc.subcore_barrier()

  issue(lin_copies(0, 0))
  wait(lin_copies(0, 0))
  issue(lin_copies(1, 1))
  issue(gat_copies(0))

  wait(gat_copies(0))
  acc = stage_a(0, zero16)
  wait(lin_copies(1, 1))
  issue(gat_copies(1))
  stage_b(0)
  issue_scatters(0)
  issue(lin_copies(2, 0))

  def pair_body(k, acc):
    i1 = 2 * k + 1
    wait(gat_copies(1))
    acc = stage_a(1, acc)
    wait(lin_copies(i1 + 1, 0))
    wait(sca_copies(0))
    issue(gat_copies(0))
    stage_b(1)
    issue_scatters(1)

    @pl.when(k < NCH // 2 - 1)
    def _():
      issue(lin_copies(i1 + 2, 1))

    i2 = 2 * k + 2
    wait(gat_copies(0))
    acc = stage_a(0, acc)

    @pl.when(k < NCH // 2 - 1)
    def _():
      wait(lin_copies(i2 + 1, 1))
      wait(sca_copies(1))
      issue(gat_copies(1))

    stage_b(0)
    issue_scatters(0)

    @pl.when(k < NCH // 2 - 1)
    def _():
      issue(lin_copies(i2 + 2, 0))

    return acc

  acc = lax.fori_loop(0, (NCH - 1) // 2, pair_body, acc)
  wait(sca_copies(1))
  wait(sca_copies(0))
  plsc.subcore_barrier()

  orow0 = cid * N + row0
  for c in range(ROW_COPIES):
    pltpu.sync_copy(out_sh.at[pl.ds(row0 + c * G, G)], xtb[0])
    pltpu.sync_copy(xtb[0], outp_hbm.at[pl.ds(orow0 + c * G, G)])
    pltpu.sync_copy(den_sh.at[pl.ds(row0 + c * G, G)], exb[0])
    pltpu.sync_copy(exb[0], denp_hbm.at[pl.ds(orow0 + c * G, G)])

  aeb[0][0, :] = acc
  for r in range(1, 8):
    aeb[0][r, :] = zero16
  pltpu.sync_copy(aeb[0].at[pl.ds(0, 8)], attp_hbm.at[pl.ds(wid * 8, 8)])


def _sc_message_pass(src, dst, aef, asrc, adst, xt):
  mesh = plsc.VectorSubcoreMesh(core_axis_name="c", subcore_axis_name="s")
  f32 = jnp.float32
  call = pl.kernel(
      _sc_body,
      out_type=[
          jax.ShapeDtypeStruct((NC * N, HEADS * HID), f32),
          jax.ShapeDtypeStruct((NC * N, 16), f32),
          jax.ShapeDtypeStruct((NW * 8, 16), f32),
      ],
      mesh=mesh,
      compiler_params=pltpu.CompilerParams(use_tc_tiling_on_sc=False),
      scratch_types=[
          [pltpu.VMEM((G + 16,), jnp.int32) for _ in range(2)],
          [pltpu.VMEM((G + 16,), jnp.int32) for _ in range(2)],
          [pltpu.VMEM((G,), jnp.int32) for _ in range(2)],
          [pltpu.VMEM((G, 16), f32) for _ in range(2)],
          [pltpu.VMEM((G, 16), f32) for _ in range(2)],
          [pltpu.VMEM((G, 16), f32) for _ in range(2)],
          [pltpu.VMEM((G, HEADS * HID), f32) for _ in range(2)],
          [pltpu.VMEM((G, 16), f32) for _ in range(2)],
          pltpu.VMEM_SHARED((N, HEADS * HID), f32),
          pltpu.VMEM_SHARED((N, 16), f32),
          [pltpu.SemaphoreType.DMA for _ in range(2)],
          [pltpu.SemaphoreType.DMA for _ in range(2)],
          [pltpu.SemaphoreType.DMA for _ in range(2)],
      ],
  )
  return call(src, dst, aef, asrc, adst, xt)


_RB = 2000


def _fin1a_body(o0_ref, o1_ref, d0_ref, d1_ref, attp_ref, bias_ref, watt_ref,
                batt_ref, g_ref, sc_ref, att_ref):
  den = d0_ref[...] + d1_ref[...] + 1e-16
  ou = o0_ref[...] + o1_ref[...]
  rden = 1.0 / den
  acc = ou[:, 0:16] * rden[:, 0:1]
  for h in range(1, HEADS):
    acc = acc + ou[:, h * 16:(h + 1) * 16] * rden[:, h:h + 1]
  g = acc * (1.0 / HEADS) + bias_ref[...]
  g = jnp.where(g > 0, g, jnp.exp(g) - 1.0)
  g_ref[...] = g
  sc = jnp.dot(g, watt_ref[...], preferred_element_type=jnp.float32)
  sc_ref[...] = sc + batt_ref[...]

  @pl.when(pl.program_id(0) == 0)
  def _():
    att_ref[...] = jnp.sum(attp_ref[...]) / (
        d0_ref[6:7, 0:1] + d1_ref[6:7, 0:1] + 1e-16)


def _finalize1a(o0, o1, d0, d1, attp, bias, watt, batt):
  return pl.pallas_call(
      _fin1a_body,
      grid=(N // _RB,),
      in_specs=[
          pl.BlockSpec((_RB, HEADS * HID), lambda i: (i, 0)),
          pl.BlockSpec((_RB, HEADS * HID), lambda i: (i, 0)),
          pl.BlockSpec((_RB, 16), lambda i: (i, 0)),
          pl.BlockSpec((_RB, 16), lambda i: (i, 0)),
          pl.BlockSpec((NW * 8, 16), lambda i: (0, 0)),
          pl.BlockSpec((1, HID), lambda i: (0, 0)),
          pl.BlockSpec((HID, 1), lambda i: (0, 0)),
          pl.BlockSpec((1, 1), lambda i: (0, 0)),
      ],
      out_specs=[
          pl.BlockSpec((_RB, HID), lambda i: (i, 0)),
          pl.BlockSpec((_RB, 1), lambda i: (i, 0)),
          pl.BlockSpec((1, 1), lambda i: (0, 0)),
      ],
      out_shape=[
          jax.ShapeDtypeStruct((N, HID), jnp.float32),
          jax.ShapeDtypeStruct((N, 1), jnp.float32),
          jax.ShapeDtypeStruct((1, 1), jnp.float32),
      ],
  )(o0, o1, d0, d1, attp, bias, watt, batt)


def _fin1b_body(g_ref, sc_ref, flat_ref):
  sc = sc_ref[...]
  m = jnp.max(sc)
  w = jnp.exp(sc - m)
  w = w / jnp.sum(w)
  flat_ref[...] = g_ref[...] * w


def _finalize1b(g, sc):
  return pl.pallas_call(
      _fin1b_body,
      out_shape=jax.ShapeDtypeStruct((N, HID), jnp.float32),
  )(g, sc)


_WB = 6400


def _fin2_body(w_ref, f_ref, bout_ref, y_ref, acc_ref):
  j = pl.program_id(0)

  @pl.when(j == 0)
  def _():
    acc_ref[...] = jnp.zeros_like(acc_ref)

  acc_ref[...] += jnp.dot(w_ref[...], f_ref[...],
                          preferred_element_type=jnp.float32)

  @pl.when(j == (N * HID) // _WB - 1)
  def _():
    y_ref[...] = acc_ref[...] + bout_ref[...]


def _finalize2(w_out, flat_col, bout):
  nblk = (N * HID) // _WB
  return pl.pallas_call(
      _fin2_body,
      grid=(nblk,),
      in_specs=[
          pl.BlockSpec((OUT_F, _WB), lambda j: (0, j)),
          pl.BlockSpec((_WB, 1), lambda j: (j, 0)),
          pl.BlockSpec((OUT_F, 1), lambda j: (0, 0)),
      ],
      out_specs=pl.BlockSpec((OUT_F, 1), lambda j: (0, 0)),
      out_shape=jax.ShapeDtypeStruct((OUT_F, 1), jnp.float32),
      scratch_shapes=[pltpu.VMEM((OUT_F, 1), jnp.float32)],
  )(w_out, flat_col, bout)


def kernel(x, edge_index, edge_attr, W_fc, b_fc, W_lin, att_src, att_dst,
           W_edge, att_edge, bias_conv, W_att, b_att, W_out, b_out):
  f32 = jnp.float32
  src = edge_index[0].astype(jnp.int32)
  dst = edge_index[1].astype(jnp.int32)

  eye8 = jnp.eye(HEADS, dtype=f32)
  pad8 = jnp.zeros((HEADS * HID, HEADS), f32)
  cs = (att_src[0][:, :, None] * eye8[:, None, :]).reshape(HEADS * HID, HEADS)
  cd = (att_dst[0][:, :, None] * eye8[:, None, :]).reshape(HEADS * HID, HEADS)
  cs = jnp.concatenate([cs, pad8], axis=1)
  cd = jnp.concatenate([cd, pad8], axis=1)
  ae_w = jnp.einsum("hjd,hj->dh", W_edge.reshape(HEADS, HID, EDGE_DIM),
                    att_edge[0]).astype(f32)
  ae_w = jnp.concatenate([ae_w, jnp.zeros((EDGE_DIM, HEADS), f32)], axis=1)

  xt, asrc, adst = _prep_nodes(
      x, W_fc.T, b_fc.reshape(1, HID), W_lin.T, cs, cd)
  aef = _prep_edges(edge_attr, ae_w)

  outp, denp, attp = _sc_message_pass(src, dst, aef, asrc, adst, xt)

  g, sc, att = _finalize1a(
      outp[:N], outp[N:], denp[:N], denp[N:], attp,
      bias_conv.reshape(1, HID), W_att.T.astype(f32), b_att.reshape(1, 1))
  flat = _finalize1b(g, sc)

  y = _finalize2(W_out, flat.reshape(N * HID, 1), b_out.reshape(OUT_F, 1))
  return (y.reshape(OUT_F), att[0, 0])


# speedup vs baseline: 60.6156x; 1.2367x over previous
import functools

import jax
import jax.numpy as jnp
from jax import lax
from jax.experimental import pallas as pl
from jax.experimental.pallas import tpu as pltpu
from jax.experimental.pallas import tpu_sc as plsc

N = 10000
E = 320000
IN_F = 128
HID = 16
HEADS = 8
EDGE_DIM = 16
OUT_F = 128

NC = 2
NS = 16
NW = NC * NS
EPT = E // NW
G = 80
NCH = EPT // G
ROW_STRIDE = 624
ROW_COPIES = 8


def _prep_nodes_body(x_ref, wfct_ref, bfc_ref, wlint_ref, cs_ref, cd_ref,
                     xt_ref, asrc_ref, adst_ref):
  h = jnp.dot(x_ref[...], wfct_ref[...], preferred_element_type=jnp.float32)
  h = h + bfc_ref[...]
  xt = jnp.dot(h, wlint_ref[...], preferred_element_type=jnp.float32)
  xt_ref[...] = xt
  asrc_ref[...] = jnp.dot(xt, cs_ref[...], preferred_element_type=jnp.float32)
  adst_ref[...] = jnp.dot(xt, cd_ref[...], preferred_element_type=jnp.float32)


def _prep_nodes(x, wfct, bfc, wlint, cs, cd):
  return pl.pallas_call(
      _prep_nodes_body,
      out_shape=[
          jax.ShapeDtypeStruct((N, HEADS * HID), jnp.float32),
          jax.ShapeDtypeStruct((N, 16), jnp.float32),
          jax.ShapeDtypeStruct((N, 16), jnp.float32),
      ],
  )(x, wfct, bfc, wlint, cs, cd)


_EB = 2000


def _prep_edges_body(ea_ref, ae_ref, out_ref):
  out_ref[...] = jnp.dot(ea_ref[...], ae_ref[...],
                         preferred_element_type=jnp.float32)


def _prep_edges(ea2, ae_blk):
  return pl.pallas_call(
      _prep_edges_body,
      grid=(E // 8 // _EB,),
      in_specs=[
          pl.BlockSpec((_EB, 128), lambda i: (i, 0)),
          pl.BlockSpec((128, 128), lambda i: (0, 0)),
      ],
      out_specs=pl.BlockSpec((_EB, 128), lambda i: (i, 0)),
      out_shape=jax.ShapeDtypeStruct((E // 8, 128), jnp.float32),
  )(ea2, ae_blk)


def _sc_body(src_hbm, dst_hbm, ae_hbm, asrc_hbm, adst_hbm, xt_hbm,
             outp_hbm, denp_hbm, attp_hbm,
             srcb, dstb, sdstb, aeb, asb, adb, xtb, exb,
             out_sh, den_sh,
             semL, semG, semS):
  cid = lax.axis_index("c")
  sid = lax.axis_index("s")
  wid = cid * NS + sid
  tile_base = wid * EPT

  io16 = lax.iota(jnp.int32, 16)
  zero16 = jnp.zeros((16,), jnp.float32)
  slope16 = jnp.full((16,), 0.2, jnp.float32)
  l0f = jnp.maximum(1.0 - io16.astype(jnp.float32),
                    jnp.zeros((16,), jnp.float32))

  def bc16(v):
    return jnp.broadcast_to(v, (16,))

  def lin_copies(ci, b):
    base = tile_base + ci * G
    return (
        pltpu.make_async_copy(src_hbm.at[pl.ds(base, G)],
                              srcb[b].at[pl.ds(0, G)], semL[b]),
        pltpu.make_async_copy(dst_hbm.at[pl.ds(base, G)],
                              dstb[b].at[pl.ds(0, G)], semL[b]),
        pltpu.make_async_copy(ae_hbm.at[pl.ds(base // 8, G // 8)], aeb[b],
                              semL[b]),
    )

  def gat_copies(b):
    si = srcb[b].at[pl.ds(0, G)]
    di = dstb[b].at[pl.ds(0, G)]
    return (
        pltpu.make_async_copy(asrc_hbm.at[si], asb[b], semG[b]),
        pltpu.make_async_copy(adst_hbm.at[di], adb[b], semG[b]),
        pltpu.make_async_copy(xt_hbm.at[si], xtb[b], semG[b]),
    )

  def sca_copies(b):
    return (
        pltpu.make_async_copy(exb[b], den_sh.at[sdstb[b]], semS[b]),
        pltpu.make_async_copy(xtb[b], out_sh.at[sdstb[b]], semS[b]),
    )

  def issue(copies):
    for c in copies:
      c.start()

  def wait(copies):
    for c in copies:
      c.wait()

  def issue_scatters(b):
    def cp(t, _):
      sdstb[b][pl.ds(t * 16, 16)] = dstb[b][pl.ds(t * 16, 16)]
      return 0
    lax.fori_loop(0, G // 16, cp, 0, unroll=True)
    pltpu.async_copy(exb[b], den_sh.at[sdstb[b]], semS[b], add=True)
    pltpu.async_copy(xtb[b], out_sh.at[sdstb[b]], semS[b], add=True)

  def stage_a(b, acc):
    def body(e, acc):
      s = asb[b][e, :]
      d = adb[b][e, :]
      a = aeb[b][lax.shift_right_logical(e, 3),
                 pl.ds(jnp.bitwise_and(e, 7) * 16, 16)]
      al = s + d + a
      al = jnp.maximum(al, slope16 * al)
      ex = jnp.exp(al)
      exb[b][e, :] = ex
      sv = srcb[b][pl.ds(e, 16)]
      dv = dstb[b][pl.ds(e, 16)]
      hit = (sv[0] == 6) & (dv[0] == 6)
      hitf = jnp.where(hit, jnp.float32(1.0), jnp.float32(0.0))
      return acc + ex * l0f * bc16(hitf)
    return lax.fori_loop(0, G, body, acc, unroll=4)

  def stage_b(b):
    def body(e, _):
      er = exb[b][e, :]
      for h in range(HEADS):
        exs = er.at[bc16(h)].get(mode="promise_in_bounds")
        sl = pl.ds(h * 16, 16)
        xtb[b][e, sl] = xtb[b][e, sl] * exs
      return 0
    lax.fori_loop(0, G, body, 0, unroll=2)

  def zero_body(i, _):
    xtb[0][lax.shift_right_logical(i, 3),
           pl.ds(jnp.bitwise_and(i, 7) * 16, 16)] = zero16
    return 0
  lax.fori_loop(0, G * HEADS, zero_body, 0, unroll=8)

  def zero_ex(e, _):
    exb[0][e, :] = zero16
    return 0
  lax.fori_loop(0, G, zero_ex, 0, unroll=4)

  row0 = sid * ROW_STRIDE
  for c in range(ROW_COPIES):
    pltpu.sync_copy(xtb[0], out_sh.at[pl.ds(row0 + c * G, G)])
    pltpu.sync_copy(exb[0], den_sh.at[pl.ds(row0 + c * G, G)])
  plsc.subcore_barrier()

  issue(lin_copies(0, 0))
  wait(lin_copies(0, 0))
  issue(lin_copies(1, 1))
  issue(gat_copies(0))

  wait(gat_copies(0))
  acc = stage_a(0, zero16)
  wait(lin_copies(1, 1))
  issue(gat_copies(1))
  stage_b(0)
  issue_scatters(0)
  issue(lin_copies(2, 0))

  def pair_body(k, acc):
    i1 = 2 * k + 1
    wait(gat_copies(1))
    acc = stage_a(1, acc)
    wait(lin_copies(i1 + 1, 0))
    wait(sca_copies(0))
    issue(gat_copies(0))
    stage_b(1)
    issue_scatters(1)

    @pl.when(k < NCH // 2 - 1)
    def _():
      issue(lin_copies(i1 + 2, 1))

    i2 = 2 * k + 2
    wait(gat_copies(0))
    acc = stage_a(0, acc)

    @pl.when(k < NCH // 2 - 1)
    def _():
      wait(lin_copies(i2 + 1, 1))
      wait(sca_copies(1))
      issue(gat_copies(1))

    stage_b(0)
    issue_scatters(0)

    @pl.when(k < NCH // 2 - 1)
    def _():
      issue(lin_copies(i2 + 2, 0))

    return acc

  acc = lax.fori_loop(0, (NCH - 1) // 2, pair_body, acc)
  wait(sca_copies(1))
  wait(sca_copies(0))
  plsc.subcore_barrier()

  orow0 = cid * N + row0
  for c in range(ROW_COPIES):
    pltpu.sync_copy(out_sh.at[pl.ds(row0 + c * G, G)], xtb[0])
    pltpu.sync_copy(xtb[0], outp_hbm.at[pl.ds(orow0 + c * G, G)])
    pltpu.sync_copy(den_sh.at[pl.ds(row0 + c * G, G)], exb[0])
    pltpu.sync_copy(exb[0], denp_hbm.at[pl.ds(orow0 + c * G, G)])

  exb[0][0, :] = acc
  for r in range(1, 8):
    exb[0][r, :] = zero16
  pltpu.sync_copy(exb[0].at[pl.ds(0, 8)], attp_hbm.at[pl.ds(wid * 8, 8)])


def _sc_message_pass(src, dst, aef, asrc, adst, xt):
  mesh = plsc.VectorSubcoreMesh(core_axis_name="c", subcore_axis_name="s")
  f32 = jnp.float32
  call = pl.kernel(
      _sc_body,
      out_type=[
          jax.ShapeDtypeStruct((NC * N, HEADS * HID), f32),
          jax.ShapeDtypeStruct((NC * N, 16), f32),
          jax.ShapeDtypeStruct((NW * 8, 16), f32),
      ],
      mesh=mesh,
      compiler_params=pltpu.CompilerParams(use_tc_tiling_on_sc=False),
      scratch_types=[
          [pltpu.VMEM((G + 16,), jnp.int32) for _ in range(2)],
          [pltpu.VMEM((G + 16,), jnp.int32) for _ in range(2)],
          [pltpu.VMEM((G,), jnp.int32) for _ in range(2)],
          [pltpu.VMEM((G // 8, 128), f32) for _ in range(2)],
          [pltpu.VMEM((G, 16), f32) for _ in range(2)],
          [pltpu.VMEM((G, 16), f32) for _ in range(2)],
          [pltpu.VMEM((G, HEADS * HID), f32) for _ in range(2)],
          [pltpu.VMEM((G, 16), f32) for _ in range(2)],
          pltpu.VMEM_SHARED((N, HEADS * HID), f32),
          pltpu.VMEM_SHARED((N, 16), f32),
          [pltpu.SemaphoreType.DMA for _ in range(2)],
          [pltpu.SemaphoreType.DMA for _ in range(2)],
          [pltpu.SemaphoreType.DMA for _ in range(2)],
      ],
  )
  return call(src, dst, aef, asrc, adst, xt)


_RB = 2000


def _fin1a_body(o0_ref, o1_ref, d0_ref, d1_ref, attp_ref, bias_ref, watt_ref,
                batt_ref, g_ref, sc_ref, att_ref):
  den = d0_ref[...] + d1_ref[...] + 1e-16
  ou = o0_ref[...] + o1_ref[...]
  rden = 1.0 / den
  acc = ou[:, 0:16] * rden[:, 0:1]
  for h in range(1, HEADS):
    acc = acc + ou[:, h * 16:(h + 1) * 16] * rden[:, h:h + 1]
  g = acc * (1.0 / HEADS) + bias_ref[...]
  g = jnp.where(g > 0, g, jnp.exp(g) - 1.0)
  g_ref[...] = g
  sc = jnp.dot(g, watt_ref[...], preferred_element_type=jnp.float32)
  sc_ref[...] = sc + batt_ref[...]

  @pl.when(pl.program_id(0) == 0)
  def _():
    att_ref[...] = jnp.sum(attp_ref[...]) / (
        d0_ref[6:7, 0:1] + d1_ref[6:7, 0:1] + 1e-16)


def _finalize1a(o0, o1, d0, d1, attp, bias, watt, batt):
  return pl.pallas_call(
      _fin1a_body,
      grid=(N // _RB,),
      in_specs=[
          pl.BlockSpec((_RB, HEADS * HID), lambda i: (i, 0)),
          pl.BlockSpec((_RB, HEADS * HID), lambda i: (i, 0)),
          pl.BlockSpec((_RB, 16), lambda i: (i, 0)),
          pl.BlockSpec((_RB, 16), lambda i: (i, 0)),
          pl.BlockSpec((NW * 8, 16), lambda i: (0, 0)),
          pl.BlockSpec((1, HID), lambda i: (0, 0)),
          pl.BlockSpec((HID, 1), lambda i: (0, 0)),
          pl.BlockSpec((1, 1), lambda i: (0, 0)),
      ],
      out_specs=[
          pl.BlockSpec((_RB, HID), lambda i: (i, 0)),
          pl.BlockSpec((_RB, 1), lambda i: (i, 0)),
          pl.BlockSpec((1, 1), lambda i: (0, 0)),
      ],
      out_shape=[
          jax.ShapeDtypeStruct((N, HID), jnp.float32),
          jax.ShapeDtypeStruct((N, 1), jnp.float32),
          jax.ShapeDtypeStruct((1, 1), jnp.float32),
      ],
  )(o0, o1, d0, d1, attp, bias, watt, batt)


def _fin1b_body(g_ref, sc_ref, flat_ref):
  sc = sc_ref[...]
  m = jnp.max(sc)
  w = jnp.exp(sc - m)
  w = w / jnp.sum(w)
  flat_ref[...] = g_ref[...] * w


def _finalize1b(g, sc):
  return pl.pallas_call(
      _fin1b_body,
      out_shape=jax.ShapeDtypeStruct((N, HID), jnp.float32),
  )(g, sc)


_WB = 6400


def _fin2_body(w_ref, f_ref, bout_ref, y_ref, acc_ref):
  j = pl.program_id(0)

  @pl.when(j == 0)
  def _():
    acc_ref[...] = jnp.zeros_like(acc_ref)

  acc_ref[...] += jnp.dot(w_ref[...], f_ref[...],
                          preferred_element_type=jnp.float32)

  @pl.when(j == (N * HID) // _WB - 1)
  def _():
    y_ref[...] = acc_ref[...] + bout_ref[...]


def _finalize2(w_out, flat_col, bout):
  nblk = (N * HID) // _WB
  return pl.pallas_call(
      _fin2_body,
      grid=(nblk,),
      in_specs=[
          pl.BlockSpec((OUT_F, _WB), lambda j: (0, j)),
          pl.BlockSpec((_WB, 1), lambda j: (j, 0)),
          pl.BlockSpec((OUT_F, 1), lambda j: (0, 0)),
      ],
      out_specs=pl.BlockSpec((OUT_F, 1), lambda j: (0, 0)),
      out_shape=jax.ShapeDtypeStruct((OUT_F, 1), jnp.float32),
      scratch_shapes=[pltpu.VMEM((OUT_F, 1), jnp.float32)],
  )(w_out, flat_col, bout)


def kernel(x, edge_index, edge_attr, W_fc, b_fc, W_lin, att_src, att_dst,
           W_edge, att_edge, bias_conv, W_att, b_att, W_out, b_out):
  f32 = jnp.float32
  src = edge_index[0].astype(jnp.int32)
  dst = edge_index[1].astype(jnp.int32)

  eye8 = jnp.eye(HEADS, dtype=f32)
  pad8 = jnp.zeros((HEADS * HID, HEADS), f32)
  cs = (att_src[0][:, :, None] * eye8[:, None, :]).reshape(HEADS * HID, HEADS)
  cd = (att_dst[0][:, :, None] * eye8[:, None, :]).reshape(HEADS * HID, HEADS)
  cs = jnp.concatenate([cs, pad8], axis=1)
  cd = jnp.concatenate([cd, pad8], axis=1)
  ae_w = jnp.einsum("hjd,hj->dh", W_edge.reshape(HEADS, HID, EDGE_DIM),
                    att_edge[0]).astype(f32)
  ae_w = jnp.concatenate([ae_w, jnp.zeros((EDGE_DIM, HEADS), f32)], axis=1)
  ae_blk = jnp.kron(jnp.eye(8, dtype=f32), ae_w)

  xt, asrc, adst = _prep_nodes(
      x, W_fc.T, b_fc.reshape(1, HID), W_lin.T, cs, cd)
  aef = _prep_edges(edge_attr.reshape(E // 8, 128), ae_blk)

  outp, denp, attp = _sc_message_pass(src, dst, aef, asrc, adst, xt)

  g, sc, att = _finalize1a(
      outp[:N], outp[N:], denp[:N], denp[N:], attp,
      bias_conv.reshape(1, HID), W_att.T.astype(f32), b_att.reshape(1, 1))
  flat = _finalize1b(g, sc)

  y = _finalize2(W_out, flat.reshape(N * HID, 1), b_out.reshape(OUT_F, 1))
  return (y.reshape(OUT_F), att[0, 0])
